# baseline (speedup 1.0000x reference)
import jax
import jax.numpy as jnp
from jax import lax
from jax.experimental import pallas as pl
from jax.experimental.pallas import tpu as pltpu
from jax.experimental.pallas import tpu_sc as plsc

BATCH = 16384
DIM = 128


_info = plsc.get_sparse_core_info()
_NC, _NS = _info.num_cores, _info.num_subcores
_NW = _NC * _NS
_CH = 256


def _pick_chunk(bpw):
    ch = min(_CH, bpw)
    while bpw % ch:
        ch //= 2
    return ch


def _make_sc_gather_body(n_rows):
    bpw = n_rows // _NW
    ch = _pick_chunk(bpw)
    n_chunks = bpw // ch

    def body(user_hbm, item_hbm, ut_hbm, it_hbm, x_out,
             idxu_v, idxi_v, bufa, bufb, gsa, gsb, ssa, ssb):
        wid = lax.axis_index("s") * _NC + lax.axis_index("c")
        base = wid * bpw
        pltpu.sync_copy(user_hbm.at[pl.ds(base, bpw)], idxu_v)
        pltpu.sync_copy(item_hbm.at[pl.ds(base, bpw)], idxi_v)
        chunks = [(idxu_v, ut_hbm, 0, c * ch) for c in range(n_chunks)]
        chunks += [(idxi_v, it_hbm, DIM, c * ch) for c in range(n_chunks)]
        bufs = [(bufa, gsa, ssa), (bufb, gsb, ssb)]
        scatters = [None, None]
        for k, (idx, tab, col, off) in enumerate(chunks):
            buf, gs, ss = bufs[k % 2]
            if scatters[k % 2] is not None:
                scatters[k % 2].wait()
            pltpu.async_copy(
                tab.at[idx.at[pl.ds(off, ch)]], buf.at[pl.ds(0, ch)], gs).wait()
            scatters[k % 2] = pltpu.async_copy(
                buf.at[pl.ds(0, ch)],
                x_out.at[pl.ds(base + off, ch), pl.ds(col, DIM)], ss)
        for s in scatters:
            if s is not None:
                s.wait()

    return body


def _sc_gather(user, item, user_table, item_table):
    n_rows = user.shape[0]
    bpw = n_rows // _NW
    ch = _pick_chunk(bpw)
    mesh = plsc.VectorSubcoreMesh(core_axis_name="c", subcore_axis_name="s")
    f = pl.kernel(
        _make_sc_gather_body(n_rows),
        mesh=mesh,
        out_type=jax.ShapeDtypeStruct((n_rows, 2 * DIM), jnp.float32),
        scratch_types=[
            pltpu.VMEM((bpw,), jnp.int32),
            pltpu.VMEM((bpw,), jnp.int32),
            pltpu.VMEM((ch, DIM), jnp.float32),
            pltpu.VMEM((ch, DIM), jnp.float32),
            pltpu.SemaphoreType.DMA,
            pltpu.SemaphoreType.DMA,
            pltpu.SemaphoreType.DMA,
            pltpu.SemaphoreType.DMA,
        ],
    )
    return f(user, item, user_table, item_table)


_BM = 4096


def _mlp_body(x_ref, w1_ref, b1_ref, w2_ref, b2_ref,
              w3_ref, b3_ref, wd_ref, bd_ref, out_ref):
    bf = jnp.bfloat16
    h = jnp.dot(x_ref[...].astype(bf), w1_ref[...].astype(bf),
                preferred_element_type=jnp.float32)
    h = jnp.maximum(h + b1_ref[...][None, :], 0.0).astype(bf)
    h = jnp.maximum(
        jnp.dot(h, w2_ref[...].astype(bf), preferred_element_type=jnp.float32)
        + b2_ref[...][None, :], 0.0).astype(bf)
    h = jnp.maximum(
        jnp.dot(h, w3_ref[...].astype(bf), preferred_element_type=jnp.float32)
        + b3_ref[...][None, :], 0.0)
    o = jnp.sum(h * wd_ref[...][None, :], axis=1, keepdims=True) + bd_ref[0]
    out_ref[...] = 1.0 / (1.0 + jnp.exp(-o))


def _mlp(x, W1, b1, W2, b2, W3, b3, Wd, bd):
    n_rows = x.shape[0]
    H1, H2, H3 = W1.shape[1], W2.shape[1], W3.shape[1]
    bf = jnp.bfloat16
    grid = (n_rows // _BM,)
    zero = lambda i: (0, 0)
    out = pl.pallas_call(
        _mlp_body,
        grid=grid,
        in_specs=[
            pl.BlockSpec((_BM, 2 * DIM), lambda i: (i, 0)),
            pl.BlockSpec((2 * DIM, H1), zero),
            pl.BlockSpec((H1,), lambda i: (0,)),
            pl.BlockSpec((H1, H2), zero),
            pl.BlockSpec((H2,), lambda i: (0,)),
            pl.BlockSpec((H2, H3), zero),
            pl.BlockSpec((H3,), lambda i: (0,)),
            pl.BlockSpec((H3,), lambda i: (0,)),
            pl.BlockSpec((1,), lambda i: (0,)),
        ],
        out_specs=pl.BlockSpec((_BM, 1), lambda i: (i, 0)),
        out_shape=jax.ShapeDtypeStruct((n_rows, 1), jnp.float32),
    )(x, W1, b1, W2, b2, W3, b3, Wd.reshape(H3), bd)
    return out


def kernel(user, item, user_table, item_table, W1, b1, W2, b2, W3, b3, Wd, bd):
    x = _sc_gather(user, item, user_table, item_table)
    out = _mlp(x, W1, b1, W2, b2, W3, b3, Wd, bd)
    return out.reshape(-1)


# speedup vs baseline: 1.0167x; 1.0167x over previous
import jax
import jax.numpy as jnp
from jax import lax
from jax.experimental import pallas as pl
from jax.experimental.pallas import tpu as pltpu
from jax.experimental.pallas import tpu_sc as plsc

BATCH = 16384
DIM = 128


_info = plsc.get_sparse_core_info()
_NC, _NS = _info.num_cores, _info.num_subcores
_NW = _NC * _NS
_CH = 256


def _make_sc_gather_body(n_rows):
    bpw = n_rows // _NW

    def body(user_hbm, item_hbm, ut_hbm, it_hbm, x_out, idx_v, rows_v, sem):
        wid = lax.axis_index("s") * _NC + lax.axis_index("c")
        base = wid * bpw
        pltpu.sync_copy(user_hbm.at[pl.ds(base, bpw)], idx_v)
        pltpu.async_copy(ut_hbm.at[idx_v], rows_v, sem).wait()
        pltpu.sync_copy(rows_v, x_out.at[pl.ds(base, bpw), pl.ds(0, DIM)])
        pltpu.sync_copy(item_hbm.at[pl.ds(base, bpw)], idx_v)
        pltpu.async_copy(it_hbm.at[idx_v], rows_v, sem).wait()
        pltpu.sync_copy(rows_v, x_out.at[pl.ds(base, bpw), pl.ds(DIM, DIM)])

    return body


def _sc_gather(user, item, user_table, item_table):
    n_rows = user.shape[0]
    bpw = n_rows // _NW
    mesh = plsc.VectorSubcoreMesh(core_axis_name="c", subcore_axis_name="s")
    f = pl.kernel(
        _make_sc_gather_body(n_rows),
        mesh=mesh,
        out_type=jax.ShapeDtypeStruct((n_rows, 2 * DIM), jnp.float32),
        scratch_types=[
            pltpu.VMEM((bpw,), jnp.int32),
            pltpu.VMEM((bpw, DIM), jnp.float32),
            pltpu.SemaphoreType.DMA,
        ],
    )
    return f(user, item, user_table, item_table)


_BM = 4096


def _mlp_body(x_ref, w1_ref, b1_ref, w2_ref, b2_ref,
              w3_ref, b3_ref, wd_ref, bd_ref, out_ref):
    bf = jnp.bfloat16
    h = jnp.dot(x_ref[...].astype(bf), w1_ref[...].astype(bf),
                preferred_element_type=jnp.float32)
    h = jnp.maximum(h + b1_ref[...][None, :], 0.0).astype(bf)
    h = jnp.maximum(
        jnp.dot(h, w2_ref[...].astype(bf), preferred_element_type=jnp.float32)
        + b2_ref[...][None, :], 0.0).astype(bf)
    h = jnp.maximum(
        jnp.dot(h, w3_ref[...].astype(bf), preferred_element_type=jnp.float32)
        + b3_ref[...][None, :], 0.0)
    o = jnp.sum(h * wd_ref[...][None, :], axis=1, keepdims=True) + bd_ref[0]
    out_ref[...] = 1.0 / (1.0 + jnp.exp(-o))


def _mlp(x, W1, b1, W2, b2, W3, b3, Wd, bd):
    n_rows = x.shape[0]
    H1, H2, H3 = W1.shape[1], W2.shape[1], W3.shape[1]
    bf = jnp.bfloat16
    grid = (n_rows // _BM,)
    zero = lambda i: (0, 0)
    out = pl.pallas_call(
        _mlp_body,
        grid=grid,
        in_specs=[
            pl.BlockSpec((_BM, 2 * DIM), lambda i: (i, 0)),
            pl.BlockSpec((2 * DIM, H1), zero),
            pl.BlockSpec((H1,), lambda i: (0,)),
            pl.BlockSpec((H1, H2), zero),
            pl.BlockSpec((H2,), lambda i: (0,)),
            pl.BlockSpec((H2, H3), zero),
            pl.BlockSpec((H3,), lambda i: (0,)),
            pl.BlockSpec((H3,), lambda i: (0,)),
            pl.BlockSpec((1,), lambda i: (0,)),
        ],
        out_specs=pl.BlockSpec((_BM, 1), lambda i: (i, 0)),
        out_shape=jax.ShapeDtypeStruct((n_rows, 1), jnp.float32),
    )(x, W1, b1, W2, b2, W3, b3, Wd.reshape(H3), bd)
    return out


def kernel(user, item, user_table, item_table, W1, b1, W2, b2, W3, b3, Wd, bd):
    x = _sc_gather(user, item, user_table, item_table)
    out = _mlp(x, W1, b1, W2, b2, W3, b3, Wd, bd)
    return out.reshape(-1)
